# baseline (speedup 1.0000x reference)
import functools

import jax
import jax.numpy as jnp
from jax import lax
from jax.experimental import pallas as pl
from jax.experimental.pallas import tpu as pltpu
from jax.experimental.pallas import tpu_sc as plsc

IN_DIM = 256
OUT_DIM = 512
MEM_LEN = 65536

NC = 2
NS = 16
NW = NC * NS
NLANE = 16
NJ = OUT_DIM // NLANE

CHUNK = 64
NBUF = 2
CHUNK_WORDS = CHUNK * OUT_DIM

SC_ROWS = 12288
SC_ROWS_PER_W = SC_ROWS // NW
SC_NCHUNKS = SC_ROWS_PER_W // CHUNK

TC_ROWS = MEM_LEN - SC_ROWS
TC_BLOCK = 4096


def _encoder_body(x_ref, mean_ref, std_ref, wt_ref, b_ref, e_ref):
    x = x_ref[...]
    mean = mean_ref[...]
    std = std_ref[...]
    new = (x - mean) / std
    new = jnp.where(std == 0, jnp.zeros_like(new), new)
    acc = jnp.dot(new, wt_ref[...], preferred_element_type=jnp.float32)
    e_ref[...] = jnp.tanh(acc + b_ref[...])


def _tc_dist_body(e_ref, mem_ref, out_ref, minacc):
    i = pl.program_id(0)

    @pl.when(i == 0)
    def _init():
        minacc[0] = jnp.float32(jnp.inf)

    e = e_ref[...]
    blk = mem_ref[...]
    dists = jnp.sum(jnp.abs(blk - e), axis=1)
    minacc[0] = jnp.minimum(minacc[0], jnp.min(dists))

    @pl.when(i == pl.num_programs(0) - 1)
    def _fin():
        out_ref[0] = minacc[0]


_GATHER_DNUMS = lax.GatherDimensionNumbers(
    offset_dims=(), collapsed_slice_dims=(0,), start_index_map=(0,)
)


def _rotate(v, idx):
    return lax.gather(
        v,
        idx[:, None],
        _GATHER_DNUMS,
        (1,),
        mode=lax.GatherScatterMode.PROMISE_IN_BOUNDS,
    )


def _rot_idx():
    lanes = lax.iota(jnp.int32, NLANE)
    return [(lanes + k) & (NLANE - 1) for k in (8, 4, 2, 1)]


def _scan_chunk(buf, e_chunks, rot_idx, minv):
    NA = 4

    def row_body(r, m):
        accs = [
            jnp.abs(buf[r, pl.ds(k * NLANE, NLANE)] - e_chunks[k])
            for k in range(NA)
        ]
        for j in range(NA, NJ):
            k = j % NA
            accs[k] = accs[k] + jnp.abs(
                buf[r, pl.ds(j * NLANE, NLANE)] - e_chunks[j]
            )
        acc = (accs[0] + accs[1]) + (accs[2] + accs[3])
        for idx in rot_idx:
            acc = acc + _rotate(acc, idx)
        return jnp.minimum(m, acc)

    return plsc.parallel_loop(0, CHUNK, carry=minv, unroll=2)(row_body)


def _sc_body(e_hbm, mem_hbm, out_hbm, e_v, bufs, outv, sems):
    wid = lax.axis_index("s") * NC + lax.axis_index("c")
    base = wid * SC_ROWS_PER_W

    pltpu.sync_copy(e_hbm, e_v)
    e_chunks = [e_v[pl.ds(j * NLANE, NLANE)] for j in range(NJ)]
    rot_idx = _rot_idx()

    def start(chunk_idx, b):
        src = mem_hbm.at[pl.ds(base + chunk_idx * CHUNK, CHUNK)]
        pltpu.make_async_copy(src, bufs[b], sems[b]).start()

    def wait(b):
        src = mem_hbm.at[pl.ds(0, CHUNK)]
        pltpu.make_async_copy(src, bufs[b], sems[b]).wait()

    for b in range(NBUF):
        start(b, b)

    def group_body(g, mv):
        c0 = g * NBUF
        for b in range(NBUF):
            wait(b)
            mv = _scan_chunk(bufs[b], e_chunks, rot_idx, mv)

            @pl.when(c0 + b + NBUF < SC_NCHUNKS)
            def _nxt():
                start(c0 + b + NBUF, b)

        return mv

    mv = lax.fori_loop(
        0, SC_NCHUNKS // NBUF, group_body, jnp.full((NLANE,), jnp.inf, jnp.float32)
    )

    outv[...] = mv
    pltpu.sync_copy(outv, out_hbm.at[wid])


def _sc_body_wrap(e_hbm, mem_hbm, out_hbm, e_v, *rest):
    bufs = rest[:NBUF]
    outv = rest[NBUF]
    sems = rest[NBUF + 1 :]
    return _sc_body(e_hbm, mem_hbm, out_hbm, e_v, bufs, outv, sems)


_sc_dist = functools.partial(
    pl.kernel,
    out_type=jax.ShapeDtypeStruct((NW, NLANE), jnp.float32),
    mesh=plsc.VectorSubcoreMesh(core_axis_name="c", subcore_axis_name="s"),
    scratch_types=(
        [pltpu.VMEM((OUT_DIM,), jnp.float32)]
        + [pltpu.VMEM((CHUNK, OUT_DIM), jnp.float32) for _ in range(NBUF)]
        + [pltpu.VMEM((NLANE,), jnp.float32)]
        + [pltpu.SemaphoreType.DMA for _ in range(NBUF)]
    ),
)(_sc_body_wrap)


def kernel(x, memory, mean, std, W_enc, b_enc):
    xf = x.reshape(1, IN_DIM)
    mean2 = mean.reshape(1, IN_DIM)
    std2 = std.reshape(1, IN_DIM)
    wt = W_enc.T
    b2 = b_enc.reshape(1, OUT_DIM)

    e = pl.pallas_call(
        _encoder_body,
        out_shape=jax.ShapeDtypeStruct((1, OUT_DIM), jnp.float32),
    )(xf, mean2, std2, wt, b2)

    sc_partials = _sc_dist(e.reshape(OUT_DIM), memory)

    tc_off = SC_ROWS // TC_BLOCK
    tc_min = pl.pallas_call(
        _tc_dist_body,
        grid=(TC_ROWS // TC_BLOCK,),
        in_specs=[
            pl.BlockSpec((1, OUT_DIM), lambda i: (0, 0)),
            pl.BlockSpec((TC_BLOCK, OUT_DIM), lambda i: (i + tc_off, 0)),
        ],
        out_specs=pl.BlockSpec(memory_space=pltpu.SMEM),
        out_shape=jax.ShapeDtypeStruct((1,), jnp.float32),
        scratch_shapes=[pltpu.SMEM((1,), jnp.float32)],
    )(e, memory)

    return jnp.minimum(jnp.min(sc_partials), tc_min[0])


# speedup vs baseline: 1.4906x; 1.4906x over previous
import jax
import jax.numpy as jnp
from jax.experimental import pallas as pl
from jax.experimental.pallas import tpu as pltpu

IN_DIM = 256
OUT_DIM = 512
MEM_LEN = 65536
BLOCK = 4096


def _dist_body(x_ref, mean_ref, std_ref, wt_ref, b_ref, mem_ref, out_ref,
               minacc, e_scr):
    i = pl.program_id(0)

    @pl.when(i == 0)
    def _init():
        x = x_ref[...]
        mean = mean_ref[...]
        std = std_ref[...]
        new = (x - mean) / std
        new = jnp.where(std == 0, jnp.zeros_like(new), new)
        acc = jnp.dot(new, wt_ref[...], preferred_element_type=jnp.float32)
        e_scr[...] = jnp.tanh(acc + b_ref[...])
        minacc[0] = jnp.float32(jnp.inf)

    e = e_scr[...]
    blk = mem_ref[...]
    dists = jnp.sum(jnp.abs(blk - e), axis=1)
    minacc[0] = jnp.minimum(minacc[0], jnp.min(dists))

    @pl.when(i == pl.num_programs(0) - 1)
    def _fin():
        out_ref[0] = minacc[0]


def kernel(x, memory, mean, std, W_enc, b_enc):
    xf = x.reshape(1, IN_DIM)
    mean2 = mean.reshape(1, IN_DIM)
    std2 = std.reshape(1, IN_DIM)
    wt = W_enc.T
    b2 = b_enc.reshape(1, OUT_DIM)

    grid = MEM_LEN // BLOCK
    out = pl.pallas_call(
        _dist_body,
        grid=(grid,),
        in_specs=[
            pl.BlockSpec((1, IN_DIM), lambda i: (0, 0)),
            pl.BlockSpec((1, IN_DIM), lambda i: (0, 0)),
            pl.BlockSpec((1, IN_DIM), lambda i: (0, 0)),
            pl.BlockSpec((IN_DIM, OUT_DIM), lambda i: (0, 0)),
            pl.BlockSpec((1, OUT_DIM), lambda i: (0, 0)),
            pl.BlockSpec((BLOCK, OUT_DIM), lambda i: (i, 0)),
        ],
        out_specs=pl.BlockSpec(memory_space=pltpu.SMEM),
        out_shape=jax.ShapeDtypeStruct((1,), jnp.float32),
        scratch_shapes=[
            pltpu.SMEM((1,), jnp.float32),
            pltpu.VMEM((1, OUT_DIM), jnp.float32),
        ],
    )(xf, mean2, std2, wt, b2, memory)
    return out[0]
